# baseline (speedup 1.0000x reference)
import jax
import jax.numpy as jnp
from jax.experimental import pallas as pl


def _split_matmul(adj_bf16, y):
    yhi = y.astype(jnp.bfloat16)
    ylo = (y - yhi.astype(jnp.float32)).astype(jnp.bfloat16)
    dn = (((0,), (0,)), ((), ()))
    hi = jax.lax.dot_general(adj_bf16, yhi, dn, preferred_element_type=jnp.float32)
    lo = jax.lax.dot_general(adj_bf16, ylo, dn, preferred_element_type=jnp.float32)
    return hi + lo


def _gcn_body(x_ref, adj_ref, W1_ref, b1_ref, W2_ref, b2_ref, out_ref):
    adj = adj_ref[...]
    deg = jnp.sum(adj, axis=0)
    dinv = jnp.where(deg > 0.0, jax.lax.rsqrt(jnp.where(deg > 0.0, deg, 1.0)), 0.0)
    dcol = dinv[:, None]
    adj_bf16 = adj.astype(jnp.bfloat16)

    xw = jnp.dot(x_ref[...], W1_ref[...], preferred_element_type=jnp.float32)
    t1 = _split_matmul(adj_bf16, xw * dcol)
    h = jnp.maximum(t1 * dcol + b1_ref[...], 0.0)

    hw = jnp.dot(h, W2_ref[...], preferred_element_type=jnp.float32)
    t2 = _split_matmul(adj_bf16, hw * dcol)
    out_ref[...] = t2 * dcol + b2_ref[...]


def kernel(x, adj, W1, b1, W2, b2):
    n = x.shape[0]
    return pl.pallas_call(
        _gcn_body,
        out_shape=jax.ShapeDtypeStruct((n, W2.shape[1]), x.dtype),
    )(x, adj, W1, b1.reshape(1, -1), W2, b2.reshape(1, -1))


# speedup vs baseline: 1.2267x; 1.2267x over previous
import jax
import jax.numpy as jnp
from jax.experimental import pallas as pl
from jax.experimental.pallas import tpu as pltpu

_N = 1024
_CHUNKS = 8
_ROWS = _N // _CHUNKS


def _gcn_body(x_ref, adj_hbm, W1_ref, b1_ref, W2_ref, b2_ref, out_ref,
              adj_vmem, sems):
    copies = [
        pltpu.make_async_copy(
            adj_hbm.at[pl.ds(c * _ROWS, _ROWS), :],
            adj_vmem.at[pl.ds(c * _ROWS, _ROWS), :],
            sems.at[c],
        )
        for c in range(_CHUNKS)
    ]
    for cp in copies:
        cp.start()

    xw = jnp.dot(x_ref[...], W1_ref[...], preferred_element_type=jnp.float32)

    ones = jnp.ones((8, _ROWS), jnp.float32)
    deg8 = jnp.zeros((8, _N), jnp.float32)
    for c in range(_CHUNKS):
        copies[c].wait()
        chunk = adj_vmem[pl.ds(c * _ROWS, _ROWS), :]
        deg8 = deg8 + jax.lax.dot_general(
            ones, chunk, (((1,), (0,)), ((), ())),
            preferred_element_type=jnp.float32,
        )
    deg = deg8[0]
    dinv = jnp.where(deg > 0.0, jax.lax.rsqrt(jnp.where(deg > 0.0, deg, 1.0)), 0.0)
    dcol = dinv[:, None]

    adj = adj_vmem[...]
    dn = (((0,), (0,)), ((), ()))
    t1 = jax.lax.dot_general(
        adj, xw * dcol, dn, preferred_element_type=jnp.float32
    )
    h = jnp.maximum(t1 * dcol + b1_ref[...], 0.0)

    hw = jnp.dot(h, W2_ref[...], preferred_element_type=jnp.float32)
    t2 = jax.lax.dot_general(
        adj, hw * dcol, dn, preferred_element_type=jnp.float32
    )
    out_ref[...] = t2 * dcol + b2_ref[...]


def kernel(x, adj, W1, b1, W2, b2):
    n = x.shape[0]
    return pl.pallas_call(
        _gcn_body,
        out_shape=jax.ShapeDtypeStruct((n, W2.shape[1]), x.dtype),
        in_specs=[
            pl.BlockSpec(memory_space=pltpu.MemorySpace.VMEM),
            pl.BlockSpec(memory_space=pltpu.MemorySpace.HBM),
            pl.BlockSpec(memory_space=pltpu.MemorySpace.VMEM),
            pl.BlockSpec(memory_space=pltpu.MemorySpace.VMEM),
            pl.BlockSpec(memory_space=pltpu.MemorySpace.VMEM),
            pl.BlockSpec(memory_space=pltpu.MemorySpace.VMEM),
        ],
        scratch_shapes=[
            pltpu.VMEM((_N, _N), jnp.float32),
            pltpu.SemaphoreType.DMA((_CHUNKS,)),
        ],
    )(x, adj, W1, b1.reshape(1, -1), W2, b2.reshape(1, -1))
